# baseline (speedup 1.0000x reference)
import functools

import jax
import jax.numpy as jnp
from jax import lax
from jax.experimental import pallas as pl
from jax.experimental.pallas import tpu as pltpu
from jax.experimental.pallas import tpu_sc as plsc

_ECONF = (
    (0.0, 0.0, 0.0, 0.0),
    (1.0, 1.0, 0.0, 0.0),
    (2.0, 2.0, 0.0, 0.0),
    (3.0, 2.0, 1.0, 0.0),
    (4.0, 2.0, 2.0, 0.0),
    (5.0, 2.0, 2.0, 1.0),
    (6.0, 2.0, 2.0, 2.0),
    (7.0, 2.0, 2.0, 3.0),
    (8.0, 2.0, 2.0, 4.0),
    (9.0, 2.0, 2.0, 5.0),
)

_ROWS = 10
_D = 256
_N = 100000
_K = 128
_NW = 32
_NBUF = 3
_LANES = 16
_FULL_CHUNKS = _N // _K
_TAIL = _N - _FULL_CHUNKS * _K
_TAIL_BASE = _FULL_CHUNKS * _K
_TAIL_WID = _FULL_CHUNKS % _NW
_ITERS = -(-_FULL_CHUNKS // _NW)
_OUTER = -(-_ITERS // _NBUF)

_mesh = plsc.VectorSubcoreMesh(core_axis_name="c", subcore_axis_name="s")


@functools.partial(
    pl.kernel,
    out_type=jax.ShapeDtypeStruct((_N, _D), jnp.float32),
    mesh=_mesh,
    compiler_params=pltpu.CompilerParams(needs_layout_passes=False),
    scratch_types=[
        pltpu.VMEM((_ROWS * _D,), jnp.float32),
        pltpu.VMEM((4, _D), jnp.float32),
        pltpu.VMEM((_NBUF, _K), jnp.int32),
        pltpu.VMEM((_NBUF, _K, _D), jnp.float32),
        [pltpu.SemaphoreType.DMA] * _NBUF,
        [pltpu.SemaphoreType.DMA] * _NBUF,
    ],
)
def _lookup_kernel(
    emb_hbm, wt_hbm, z_hbm, out_hbm, tbl_v, wt_v, idx_v, rows_v, zsem, wsem
):
    wid = lax.axis_index("s") * 2 + lax.axis_index("c")

    pltpu.sync_copy(emb_hbm, tbl_v)
    pltpu.sync_copy(wt_hbm, wt_v)
    for r in range(_ROWS):
        coefs = [(k, c) for k, c in enumerate(_ECONF[r]) if c != 0.0]
        if not coefs:
            continue
        for j in range(_D // _LANES):
            sl = pl.ds(r * _D + j * _LANES, _LANES)
            acc = tbl_v[sl]
            for k, c in coefs:
                acc = acc + c * wt_v[k, pl.ds(j * _LANES, _LANES)]
            tbl_v[sl] = acc

    def zcopy(c, q):
        base = pl.multiple_of(c * _K, _K)
        return pltpu.make_async_copy(
            z_hbm.at[pl.ds(base, _K)], idx_v.at[q], zsem[q]
        )

    def wcopy(c, q):
        base = pl.multiple_of(c * _K, _K)
        return pltpu.make_async_copy(
            rows_v.at[q], out_hbm.at[pl.ds(base, _K)], wsem[q]
        )

    def assemble(q, n_rows):
        for g in range(n_rows // _LANES):
            zv = idx_v.at[q][pl.ds(g * _LANES, _LANES)]
            addr0 = zv * _D
            rowv = lax.iota(jnp.int32, _LANES) + (g * _LANES)
            colv0 = jnp.zeros((_LANES,), jnp.int32)

            def body(s, carry, q=q, rowv=rowv):
                addr, colv = carry
                for _ in range(_LANES):
                    vals = plsc.load_gather(tbl_v, [addr])
                    plsc.store_scatter(rows_v.at[q], [rowv, colv], vals)
                    addr = addr + 1
                    colv = colv + 1
                return addr, colv

            lax.fori_loop(0, _D // _LANES, body, (addr0, colv0))

    for q in range(_NBUF):
        c0 = wid + _NW * q

        @pl.when(c0 < _FULL_CHUNKS)
        def _(c0=c0, q=q):
            zcopy(c0, q).start()

    def outer(it, carry):
        for q in range(_NBUF):
            t = it * _NBUF + q
            c = wid + _NW * t
            cp = c - _NW * _NBUF

            @pl.when((it > 0) & (cp < _FULL_CHUNKS))
            def _(cp=cp, q=q):
                wcopy(cp, q).wait()

            @pl.when(c < _FULL_CHUNKS)
            def _(c=c, q=q):
                zcopy(c, q).wait()
                assemble(q, _K)
                wcopy(c, q).start()

            cn = c + _NW * _NBUF

            @pl.when(cn < _FULL_CHUNKS)
            def _(cn=cn, q=q):
                zcopy(cn, q).start()

        return carry

    lax.fori_loop(0, _OUTER, outer, 0)

    for t in range(_OUTER * _NBUF - _NBUF, _OUTER * _NBUF):
        c = wid + _NW * t

        @pl.when(c < _FULL_CHUNKS)
        def _(c=c, t=t):
            wcopy(c, t % _NBUF).wait()

    @pl.when(wid == _TAIL_WID)
    def _():
        pltpu.sync_copy(
            z_hbm.at[pl.ds(_TAIL_BASE, _TAIL)], idx_v.at[0].at[pl.ds(0, _TAIL)]
        )
        assemble(0, _TAIL)
        pltpu.sync_copy(
            rows_v.at[0].at[pl.ds(0, _TAIL)],
            out_hbm.at[pl.ds(_TAIL_BASE, _TAIL)],
        )


def kernel(z, element_embedding, W):
    return _lookup_kernel(
        element_embedding.reshape(-1), W.T, z.astype(jnp.int32)
    )


# speedup vs baseline: 9.1167x; 9.1167x over previous
import functools

import jax
import jax.numpy as jnp
from jax import lax
from jax.experimental import pallas as pl
from jax.experimental.pallas import tpu as pltpu
from jax.experimental.pallas import tpu_sc as plsc

_ECONF = (
    (0.0, 0.0, 0.0, 0.0),
    (1.0, 1.0, 0.0, 0.0),
    (2.0, 2.0, 0.0, 0.0),
    (3.0, 2.0, 1.0, 0.0),
    (4.0, 2.0, 2.0, 0.0),
    (5.0, 2.0, 2.0, 1.0),
    (6.0, 2.0, 2.0, 2.0),
    (7.0, 2.0, 2.0, 3.0),
    (8.0, 2.0, 2.0, 4.0),
    (9.0, 2.0, 2.0, 5.0),
)

_ROWS = 10
_D = 256
_N = 100000
_K = 128
_REP = 512
_NW = 32
_NBUF = 3
_NIB = 6
_LANES = 16
_FULL_CHUNKS = _N // _K
_TAIL = _N - _FULL_CHUNKS * _K
_TAIL_BASE = _FULL_CHUNKS * _K
_TAIL_WID = _FULL_CHUNKS % _NW
_ITERS = -(-_FULL_CHUNKS // _NW)


def _table_body(econf_ref, emb_ref, wt_ref, out_ref):
    acc = emb_ref[...]
    for k in range(4):
        acc = acc + econf_ref[:, k : k + 1] * wt_ref[k : k + 1, :]
    out_ref[...] = jnp.broadcast_to(acc[None], (_REP, _ROWS, _D))


def _build_table(econf, emb, wt):
    return pl.pallas_call(
        _table_body,
        out_shape=jax.ShapeDtypeStruct((_REP, _ROWS, _D), jnp.float32),
    )(econf, emb, wt)


_mesh = plsc.VectorSubcoreMesh(core_axis_name="c", subcore_axis_name="s")


@functools.partial(
    pl.kernel,
    out_type=jax.ShapeDtypeStruct((_N, _D), jnp.float32),
    mesh=_mesh,
    scratch_types=[
        pltpu.VMEM((_NIB, _K), jnp.int32),
        pltpu.VMEM((_NBUF, _K, _D), jnp.float32),
        [pltpu.SemaphoreType.DMA] * _NIB,
        [pltpu.SemaphoreType.DMA] * _NBUF,
        [pltpu.SemaphoreType.DMA] * _NBUF,
    ],
)
def _gather_kernel(table_hbm, z_hbm, out_hbm, idx_v, rows_v, zsem, gsem, wsem):
    wid = lax.axis_index("s") * 2 + lax.axis_index("c")

    def chunk_of(t):
        return wid + _NW * t

    def zcopy(t):
        i = t % _NIB
        base = pl.multiple_of(chunk_of(t) * _K, _K)
        return pltpu.make_async_copy(
            z_hbm.at[pl.ds(base, _K)], idx_v.at[i], zsem[i]
        )

    def wcopy(t):
        b = t % _NBUF
        base = pl.multiple_of(chunk_of(t) * _K, _K)
        return pltpu.make_async_copy(
            rows_v.at[b], out_hbm.at[pl.ds(base, _K)], wsem[b]
        )

    def gcopy(t):
        return pltpu.make_async_copy(
            table_hbm.at[idx_v.at[t % _NIB]],
            rows_v.at[t % _NBUF],
            gsem[t % _NBUF],
        )

    def spread(i, n, t):
        rot = (t % 4) * _K if not isinstance(t, int) else ((t % 4) * _K)
        for j in range(n // _LANES):
            sl = pl.ds(j * _LANES, _LANES)
            off = (lax.iota(jnp.int32, _LANES) + (j * _LANES + rot)) * _ROWS
            idx_v.at[i][sl] = idx_v.at[i][sl] + off

    def prefetch(t):
        if t >= _ITERS:
            return

        @pl.when(chunk_of(t) < _FULL_CHUNKS)
        def _():
            zcopy(t).start()

    def fire(t):
        @pl.when(chunk_of(t) < _FULL_CHUNKS)
        def _():
            if t >= _NBUF:
                wcopy(t - _NBUF).wait()
            zcopy(t).wait()
            spread(t % _NIB, _K, t)
            gcopy(t).start()

    def drain(t):
        @pl.when(chunk_of(t) < _FULL_CHUNKS)
        def _():
            gcopy(t).wait()
            wcopy(t).start()

    def finish(t):
        @pl.when(chunk_of(t) < _FULL_CHUNKS)
        def _():
            wcopy(t).wait()

    for t in range(_NBUF):
        prefetch(t)
    fire(0)
    for t in range(1, _ITERS):
        prefetch(t + _NBUF - 1)
        fire(t)
        drain(t - 1)
    drain(_ITERS - 1)
    for t in range(max(0, _ITERS - _NBUF), _ITERS):
        finish(t)

    @pl.when(wid == _TAIL_WID)
    def _():
        pltpu.sync_copy(
            z_hbm.at[pl.ds(_TAIL_BASE, _TAIL)], idx_v.at[0].at[pl.ds(0, _TAIL)]
        )
        spread(0, _TAIL, 0)
        pltpu.async_copy(
            table_hbm.at[idx_v.at[0].at[pl.ds(0, _TAIL)]],
            rows_v.at[0].at[pl.ds(0, _TAIL)],
            gsem[0],
        ).wait()
        pltpu.sync_copy(
            rows_v.at[0].at[pl.ds(0, _TAIL)],
            out_hbm.at[pl.ds(_TAIL_BASE, _TAIL)],
        )


def kernel(z, element_embedding, W):
    econf = jnp.asarray(_ECONF, dtype=jnp.float32)
    table = _build_table(econf, element_embedding, W.T)
    table = table.reshape(_REP * _ROWS, _D)
    return _gather_kernel(table, z.astype(jnp.int32))
